# baseline (speedup 1.0000x reference)
import functools

import jax
import jax.numpy as jnp
from jax import lax
from jax.experimental import pallas as pl
from jax.experimental.pallas import tpu as pltpu
from jax.experimental.pallas import tpu_sc as plsc

N = 10000
E = 320000
D = 128

NC = 2
NS = 16
NW = NC * NS

NP = 10240
RPT = NP // NS
C = 128
EPAD = 327680
NCHUNK = EPAD // (NW * C)
EPP = NCHUNK * C
HALF = NCHUNK // 2


@functools.lru_cache(maxsize=None)
def _sc_kernels():
    mesh = plsc.VectorSubcoreMesh(core_axis_name="c", subcore_axis_name="s",
                                  num_cores=NC, num_subcores=NS)

    @functools.partial(
        pl.kernel,
        out_type=jax.ShapeDtypeStruct((4, NP), jnp.float32),
        mesh=mesh,
        scratch_types=[
            pltpu.VMEM((NCHUNK, C), jnp.int32),
            pltpu.VMEM((NCHUNK, C), jnp.int32),
            pltpu.VMEM((C,), jnp.float32),
            pltpu.VMEM_SHARED((NP,), jnp.float32),
            pltpu.VMEM_SHARED((NP,), jnp.float32),
        ],
    )
    def deg_kernel(src_hbm, dst_hbm, zero1_hbm, out_hbm, idx_s, idx_d, ones_v,
                   dego_sp, degi_sp):
        cid = lax.axis_index("c")
        sid = lax.axis_index("s")
        wid = sid * NC + cid

        for i in range(C // 16):
            ones_v[pl.ds(i * 16, 16)] = jnp.ones((16,), jnp.float32)

        pltpu.sync_copy(zero1_hbm.at[pl.ds(sid * RPT, RPT)],
                        dego_sp.at[pl.ds(sid * RPT, RPT)])
        pltpu.sync_copy(zero1_hbm.at[pl.ds(sid * RPT, RPT)],
                        degi_sp.at[pl.ds(sid * RPT, RPT)])
        pltpu.sync_copy(src_hbm.at[wid], idx_s)
        pltpu.sync_copy(dst_hbm.at[wid], idx_d)
        plsc.subcore_barrier()

        def chunk(g, carry):
            pltpu.sync_copy(ones_v, dego_sp.at[idx_s.at[g]], add=True)
            pltpu.sync_copy(ones_v, degi_sp.at[idx_d.at[g]], add=True)
            return carry

        lax.fori_loop(0, NCHUNK, chunk, 0)
        plsc.subcore_barrier()

        pltpu.sync_copy(dego_sp.at[pl.ds(sid * RPT, RPT)],
                        out_hbm.at[2 * cid, pl.ds(sid * RPT, RPT)])
        pltpu.sync_copy(degi_sp.at[pl.ds(sid * RPT, RPT)],
                        out_hbm.at[2 * cid + 1, pl.ds(sid * RPT, RPT)])

    @functools.partial(
        pl.kernel,
        out_type=jax.ShapeDtypeStruct((NC, NP, D), jnp.float32),
        mesh=mesh,
        scratch_types=[
            pltpu.VMEM((NCHUNK, C), jnp.int32),
            pltpu.VMEM((NCHUNK, C), jnp.int32),
            pltpu.VMEM((C, D), jnp.float32),
            pltpu.VMEM_SHARED((NP, D), jnp.float32),
            pltpu.SemaphoreType.DMA,
        ],
    )
    def msg_kernel(t_hbm, src_hbm, dst_hbm, zero_hbm, out_hbm, idx_s, idx_d,
                   rows, agg_sp, gsem):
        cid = lax.axis_index("c")
        sid = lax.axis_index("s")
        wid = sid * NC + cid

        pltpu.sync_copy(zero_hbm.at[pl.ds(sid * RPT, RPT)],
                        agg_sp.at[pl.ds(sid * RPT, RPT)])
        pltpu.sync_copy(src_hbm.at[wid], idx_s)
        pltpu.sync_copy(dst_hbm.at[wid], idx_d)
        plsc.subcore_barrier()

        def chunk(g, carry):
            pltpu.async_copy(t_hbm.at[idx_s.at[g]], rows, gsem).wait()
            pltpu.sync_copy(rows, agg_sp.at[idx_d.at[g]], add=True)
            return carry

        lax.fori_loop(0, NCHUNK, chunk, 0)
        plsc.subcore_barrier()

        pltpu.sync_copy(agg_sp.at[pl.ds(sid * RPT, RPT)],
                        out_hbm.at[cid, pl.ds(sid * RPT, RPT)])

    return deg_kernel, msg_kernel


_BLK = 1024


def _prep_body(x_ref, deg_ref, t1_ref, ro_ref, ri_ref):
    d_out = deg_ref[0, :] + deg_ref[2, :]
    d_in = deg_ref[1, :] + deg_ref[3, :]
    ro = lax.rsqrt(jnp.maximum(d_out, 1.0))[:, None]
    ri = lax.rsqrt(jnp.maximum(d_in, 1.0))[:, None]
    ro_ref[...] = ro
    ri_ref[...] = ri
    t1_ref[...] = x_ref[...] * ro


def _prep(x_pad, deg):
    return pl.pallas_call(
        _prep_body,
        grid=(NP // _BLK,),
        in_specs=[
            pl.BlockSpec((_BLK, D), lambda i: (i, 0)),
            pl.BlockSpec((4, _BLK), lambda i: (0, i)),
        ],
        out_specs=[
            pl.BlockSpec((_BLK, D), lambda i: (i, 0)),
            pl.BlockSpec((_BLK, 1), lambda i: (i, 0)),
            pl.BlockSpec((_BLK, 1), lambda i: (i, 0)),
        ],
        out_shape=[
            jax.ShapeDtypeStruct((NP, D), jnp.float32),
            jax.ShapeDtypeStruct((NP, 1), jnp.float32),
            jax.ShapeDtypeStruct((NP, 1), jnp.float32),
        ],
    )(x_pad, deg)


def _layer_body(relu, post_scale, agg_ref, ri_ref, ro_ref, w_ref, b_ref,
                out_ref):
    a = (agg_ref[0] + agg_ref[1]) * ri_ref[...]
    y = jnp.dot(a, w_ref[...], preferred_element_type=jnp.float32)
    y = y + b_ref[...]
    if relu:
        y = jnp.maximum(y, 0.0)
    if post_scale:
        y = y * ro_ref[...]
    out_ref[...] = y


def _layer(agg, ri, ro, w, b, relu, post_scale):
    return pl.pallas_call(
        functools.partial(_layer_body, relu, post_scale),
        grid=(NP // _BLK,),
        in_specs=[
            pl.BlockSpec((NC, _BLK, D), lambda i: (0, i, 0)),
            pl.BlockSpec((_BLK, 1), lambda i: (i, 0)),
            pl.BlockSpec((_BLK, 1), lambda i: (i, 0)),
            pl.BlockSpec((D, D), lambda i: (0, 0)),
            pl.BlockSpec((1, D), lambda i: (0, 0)),
        ],
        out_specs=pl.BlockSpec((_BLK, D), lambda i: (i, 0)),
        out_shape=jax.ShapeDtypeStruct((NP, D), jnp.float32),
    )(agg, ri, ro, w, b.reshape(1, D))


def kernel(x, edge_index, W1, b1, W2, b2):
    src = edge_index[0].astype(jnp.int32)
    dst = edge_index[1].astype(jnp.int32)

    pad = EPAD - E
    src_p = jnp.concatenate([src, jnp.zeros((pad,), jnp.int32)])
    dst_p = jnp.concatenate(
        [dst, N + (jnp.arange(pad, dtype=jnp.int32) % (NP - N))])
    src_r = src_p.reshape(NW, NCHUNK, C)
    dst_r = dst_p.reshape(NW, NCHUNK, C)
    src_m = src_r + (jnp.arange(NW, dtype=jnp.int32) % NC)[:, None, None] * NP

    x_pad = jnp.pad(x, ((0, NP - N), (0, 0)))
    zero2 = jnp.zeros((NP, D), jnp.float32)
    zero1 = jnp.zeros((NP,), jnp.float32)

    deg_kernel, msg_kernel = _sc_kernels()
    deg = deg_kernel(src_r, dst_r, zero1)
    t1, ro, ri = _prep(x_pad, deg)
    t1d = jnp.concatenate([t1, t1], axis=0)
    agg1 = msg_kernel(t1d, src_m, dst_r, zero2)
    t2 = _layer(agg1, ri, ro, W1, b1, relu=True, post_scale=True)
    t2d = jnp.concatenate([t2, t2], axis=0)
    agg2 = msg_kernel(t2d, src_m, dst_r, zero2)
    out = _layer(agg2, ri, ro, W2, b2, relu=False, post_scale=False)
    return out[:N]


# speedup vs baseline: 1.4110x; 1.4110x over previous
import functools

import jax
import jax.numpy as jnp
from jax import lax
from jax.experimental import pallas as pl
from jax.experimental.pallas import tpu as pltpu
from jax.experimental.pallas import tpu_sc as plsc

N = 10000
E = 320000
D = 128

NC = 2
NS = 16
NW = NC * NS

NP = 10240
RPT = NP // NS
C = 128
EPAD = 323584
NCHUNK = EPAD // (NW * C)
EPP = NCHUNK * C


@functools.lru_cache(maxsize=None)
def _sc_kernels():
    mesh = plsc.VectorSubcoreMesh(core_axis_name="c", subcore_axis_name="s",
                                  num_cores=NC, num_subcores=NS)

    @functools.partial(
        pl.kernel,
        out_type=jax.ShapeDtypeStruct((4, NP), jnp.float32),
        mesh=mesh,
        scratch_types=[
            pltpu.VMEM((NCHUNK, C), jnp.int32),
            pltpu.VMEM((NCHUNK, C), jnp.int32),
            pltpu.VMEM((C,), jnp.float32),
            pltpu.VMEM_SHARED((NP,), jnp.float32),
            pltpu.VMEM_SHARED((NP,), jnp.float32),
        ],
    )
    def deg_kernel(src_hbm, dst_hbm, zero1_hbm, out_hbm, idx_s, idx_d, ones_v,
                   dego_sp, degi_sp):
        cid = lax.axis_index("c")
        sid = lax.axis_index("s")
        wid = sid * NC + cid

        for i in range(C // 16):
            ones_v[pl.ds(i * 16, 16)] = jnp.ones((16,), jnp.float32)

        pltpu.sync_copy(zero1_hbm.at[pl.ds(sid * RPT, RPT)],
                        dego_sp.at[pl.ds(sid * RPT, RPT)])
        pltpu.sync_copy(zero1_hbm.at[pl.ds(sid * RPT, RPT)],
                        degi_sp.at[pl.ds(sid * RPT, RPT)])
        pltpu.sync_copy(src_hbm.at[wid], idx_s)
        pltpu.sync_copy(dst_hbm.at[wid], idx_d)
        plsc.subcore_barrier()

        def chunk(g, carry):
            pltpu.sync_copy(ones_v, dego_sp.at[idx_s.at[g]], add=True)
            pltpu.sync_copy(ones_v, degi_sp.at[idx_d.at[g]], add=True)
            return carry

        lax.fori_loop(0, NCHUNK, chunk, 0)
        plsc.subcore_barrier()

        pltpu.sync_copy(dego_sp.at[pl.ds(sid * RPT, RPT)],
                        out_hbm.at[2 * cid, pl.ds(sid * RPT, RPT)])
        pltpu.sync_copy(degi_sp.at[pl.ds(sid * RPT, RPT)],
                        out_hbm.at[2 * cid + 1, pl.ds(sid * RPT, RPT)])

    @functools.partial(
        pl.kernel,
        out_type=jax.ShapeDtypeStruct((NC, NP, D), jnp.float32),
        mesh=mesh,
        scratch_types=[
            pltpu.VMEM((NCHUNK, C), jnp.int32),
            pltpu.VMEM((NCHUNK, C), jnp.int32),
            pltpu.VMEM((C, D), jnp.float32),
            pltpu.VMEM_SHARED((NP, D), jnp.float32),
            pltpu.SemaphoreType.DMA,
        ],
    )
    def msg_kernel(t_hbm, src_hbm, dst_hbm, zero_hbm, out_hbm, idx_s, idx_d,
                   rows, agg_sp, gsem):
        cid = lax.axis_index("c")
        sid = lax.axis_index("s")
        wid = sid * NC + cid

        pltpu.sync_copy(zero_hbm.at[pl.ds(sid * RPT, RPT)],
                        agg_sp.at[pl.ds(sid * RPT, RPT)])
        pltpu.sync_copy(src_hbm.at[wid], idx_s)
        pltpu.sync_copy(dst_hbm.at[wid], idx_d)
        plsc.subcore_barrier()

        def chunk(g, carry):
            pltpu.async_copy(t_hbm.at[idx_s.at[g]], rows, gsem).wait()
            pltpu.sync_copy(rows, agg_sp.at[idx_d.at[g]], add=True)
            return carry

        lax.fori_loop(0, NCHUNK, chunk, 0)
        plsc.subcore_barrier()

        pltpu.sync_copy(agg_sp.at[pl.ds(sid * RPT, RPT)],
                        out_hbm.at[cid, pl.ds(sid * RPT, RPT)])

    return deg_kernel, msg_kernel


_BLK = 1024


def _prep_body(x_ref, deg_ref, t1_ref, ro_ref, ri_ref):
    d_out = deg_ref[0, :] + deg_ref[2, :]
    d_in = deg_ref[1, :] + deg_ref[3, :]
    ro = lax.rsqrt(jnp.maximum(d_out, 1.0))[:, None]
    ri = lax.rsqrt(jnp.maximum(d_in, 1.0))[:, None]
    ro_ref[...] = ro
    ri_ref[...] = ri
    t1_ref[...] = x_ref[...] * ro


def _prep(x_pad, deg):
    return pl.pallas_call(
        _prep_body,
        grid=(NP // _BLK,),
        in_specs=[
            pl.BlockSpec((_BLK, D), lambda i: (i, 0)),
            pl.BlockSpec((4, _BLK), lambda i: (0, i)),
        ],
        out_specs=[
            pl.BlockSpec((_BLK, D), lambda i: (i, 0)),
            pl.BlockSpec((_BLK, 1), lambda i: (i, 0)),
            pl.BlockSpec((_BLK, 1), lambda i: (i, 0)),
        ],
        out_shape=[
            jax.ShapeDtypeStruct((NP, D), jnp.float32),
            jax.ShapeDtypeStruct((NP, 1), jnp.float32),
            jax.ShapeDtypeStruct((NP, 1), jnp.float32),
        ],
    )(x_pad, deg)


def _layer_body(relu, post_scale, agg_ref, ri_ref, ro_ref, w_ref, b_ref,
                out_ref):
    a = (agg_ref[0] + agg_ref[1]) * ri_ref[...]
    y = jnp.dot(a, w_ref[...], preferred_element_type=jnp.float32)
    y = y + b_ref[...]
    if relu:
        y = jnp.maximum(y, 0.0)
    if post_scale:
        y = y * ro_ref[...]
    out_ref[...] = y


def _layer(agg, ri, ro, w, b, relu, post_scale):
    return pl.pallas_call(
        functools.partial(_layer_body, relu, post_scale),
        grid=(NP // _BLK,),
        in_specs=[
            pl.BlockSpec((NC, _BLK, D), lambda i: (0, i, 0)),
            pl.BlockSpec((_BLK, 1), lambda i: (i, 0)),
            pl.BlockSpec((_BLK, 1), lambda i: (i, 0)),
            pl.BlockSpec((D, D), lambda i: (0, 0)),
            pl.BlockSpec((1, D), lambda i: (0, 0)),
        ],
        out_specs=pl.BlockSpec((_BLK, D), lambda i: (i, 0)),
        out_shape=jax.ShapeDtypeStruct((NP, D), jnp.float32),
    )(agg, ri, ro, w, b.reshape(1, D))


def kernel(x, edge_index, W1, b1, W2, b2):
    src = edge_index[0].astype(jnp.int32)
    dst = edge_index[1].astype(jnp.int32)

    pad = EPAD - E
    src_p = jnp.concatenate([src, jnp.zeros((pad,), jnp.int32)])
    dst_p = jnp.concatenate(
        [dst, N + (jnp.arange(pad, dtype=jnp.int32) % (NP - N))])
    src_r = src_p.reshape(NW, NCHUNK, C)
    dst_r = dst_p.reshape(NW, NCHUNK, C)

    x_pad = jnp.pad(x, ((0, NP - N), (0, 0)))
    zero2 = jnp.zeros((NP, D), jnp.float32)
    zero1 = jnp.zeros((NP,), jnp.float32)

    deg_kernel, msg_kernel = _sc_kernels()
    deg = deg_kernel(src_r, dst_r, zero1)
    t1, ro, ri = _prep(x_pad, deg)
    agg1 = msg_kernel(t1, src_r, dst_r, zero2)
    t2 = _layer(agg1, ri, ro, W1, b1, relu=True, post_scale=True)
    agg2 = msg_kernel(t2, src_r, dst_r, zero2)
    out = _layer(agg2, ri, ro, W2, b2, relu=False, post_scale=False)
    return out[:N]
